# baseline (device time: 17557 ns/iter reference)
import jax
import jax.numpy as jnp
from jax import lax
from jax.experimental import pallas as pl
from jax.experimental.pallas import tpu as pltpu

C = 8


def kernel(x):
    m_per, n = x.shape
    m_glob = 2 * m_per
    n_per = n // 2
    half = m_per // 2
    rows = half // C

    def body(
        x_ref, out_ref,
        stage_src, send_buf, rx_buf, ry_buf, rx_f32, ry_f32,
        in_sems, local_sem, sx, rx_sem, sy, ry_sem, srx, sry,
    ):
        mx = lax.axis_index("x")
        my = lax.axis_index("y")
        px = 1 - mx
        py = 1 - my

        barrier_sem = pltpu.get_barrier_semaphore()
        pl.semaphore_signal(
            barrier_sem, inc=1,
            device_id=(px, my), device_id_type=pl.DeviceIdType.MESH,
        )
        pl.semaphore_signal(
            barrier_sem, inc=1,
            device_id=(mx, py), device_id_type=pl.DeviceIdType.MESH,
        )

        def in_copy(i):
            return pltpu.make_async_copy(
                x_ref.at[pl.ds(my * half + i * rows, rows), pl.ds(px * n_per, n_per)],
                stage_src.at[i],
                in_sems.at[i],
            )

        for i in range(C):
            in_copy(i).start()
        local_copy = pltpu.make_async_copy(
            x_ref.at[:, pl.ds(mx * n_per, n_per)],
            out_ref.at[pl.ds(mx * m_per, m_per), :],
            local_sem.at[0],
        )
        local_copy.start()

        pl.semaphore_wait(barrier_sem, 2)

        def x_rdma(i):
            return pltpu.make_async_remote_copy(
                src_ref=send_buf.at[i],
                dst_ref=rx_buf.at[i],
                send_sem=sx.at[i],
                recv_sem=rx_sem.at[i],
                device_id=(px, my),
                device_id_type=pl.DeviceIdType.MESH,
            )

        def y_rdma(i):
            return pltpu.make_async_remote_copy(
                src_ref=rx_buf.at[i],
                dst_ref=ry_buf.at[i],
                send_sem=sy.at[i],
                recv_sem=ry_sem.at[i],
                device_id=(mx, py),
                device_id_type=pl.DeviceIdType.MESH,
            )

        for i in range(C):
            in_copy(i).wait()
            send_buf[i, :, :] = stage_src[i].astype(jnp.bfloat16)
            x_rdma(i).start()

        for i in range(C):
            x_rdma(i).wait_recv()
            y_rdma(i).start()
            rx_f32[i, :, :] = rx_buf[i].astype(jnp.float32)
            pltpu.make_async_copy(
                rx_f32.at[i],
                out_ref.at[pl.ds(px * m_per + my * half + i * rows, rows), :],
                srx.at[i],
            ).start()

        for i in range(C):
            y_rdma(i).wait_recv()
            ry_f32[i, :, :] = ry_buf[i].astype(jnp.float32)
            pltpu.make_async_copy(
                ry_f32.at[i],
                out_ref.at[pl.ds(px * m_per + py * half + i * rows, rows), :],
                sry.at[i],
            ).start()

        local_copy.wait()
        for i in range(C):
            x_rdma(i).wait_send()
            y_rdma(i).wait_send()
            pltpu.make_async_copy(
                rx_f32.at[i],
                out_ref.at[pl.ds(px * m_per + my * half + i * rows, rows), :],
                srx.at[i],
            ).wait()
            pltpu.make_async_copy(
                ry_f32.at[i],
                out_ref.at[pl.ds(px * m_per + py * half + i * rows, rows), :],
                sry.at[i],
            ).wait()

    return pl.pallas_call(
        body,
        out_shape=jax.ShapeDtypeStruct((m_glob, n_per), x.dtype),
        in_specs=[pl.BlockSpec(memory_space=pl.ANY)],
        out_specs=pl.BlockSpec(memory_space=pl.ANY),
        scratch_shapes=[
            pltpu.VMEM((C, rows, n_per), jnp.float32),
            pltpu.VMEM((C, rows, n_per), jnp.bfloat16),
            pltpu.VMEM((C, rows, n_per), jnp.bfloat16),
            pltpu.VMEM((C, rows, n_per), jnp.bfloat16),
            pltpu.VMEM((C, rows, n_per), jnp.float32),
            pltpu.VMEM((C, rows, n_per), jnp.float32),
            pltpu.SemaphoreType.DMA((C,)),
            pltpu.SemaphoreType.DMA((1,)),
            pltpu.SemaphoreType.DMA((C,)),
            pltpu.SemaphoreType.DMA((C,)),
            pltpu.SemaphoreType.DMA((C,)),
            pltpu.SemaphoreType.DMA((C,)),
            pltpu.SemaphoreType.DMA((C,)),
            pltpu.SemaphoreType.DMA((C,)),
        ],
        compiler_params=pltpu.CompilerParams(collective_id=0),
    )(x)


# device time: 16210 ns/iter; 1.0831x vs baseline; 1.0831x over previous
import jax
import jax.numpy as jnp
from jax import lax
from jax.experimental import pallas as pl
from jax.experimental.pallas import tpu as pltpu

R = 32
N_F = 13
N_D = 6
N_X = N_F + N_D
F_ROWS = R * N_F
D_BASE = 2 * F_ROWS


def kernel(x):
    m_per, n = x.shape
    m_glob = 2 * m_per
    n_per = n // 2
    assert 2 * F_ROWS + R * N_D == m_per

    def body(x_ref, out_ref, send_buf, rx_buf, ry_buf, sx, rx_sem, sy, ry_sem):
        mx = lax.axis_index("x")
        my = lax.axis_index("y")
        px = 1 - mx
        py = 1 - my

        barrier_sem = pltpu.get_barrier_semaphore()
        pl.semaphore_signal(
            barrier_sem, inc=1,
            device_id=(px, my), device_id_type=pl.DeviceIdType.MESH,
        )
        pl.semaphore_signal(
            barrier_sem, inc=1,
            device_id=(mx, py), device_id_type=pl.DeviceIdType.MESH,
        )
        pl.semaphore_wait(barrier_sem, 2)

        def x_rdma(i):
            return pltpu.make_async_remote_copy(
                src_ref=send_buf.at[i],
                dst_ref=rx_buf.at[i],
                send_sem=sx.at[i],
                recv_sem=rx_sem.at[i],
                device_id=(px, my),
                device_id_type=pl.DeviceIdType.MESH,
            )

        def y_rdma(i):
            return pltpu.make_async_remote_copy(
                src_ref=rx_buf.at[i],
                dst_ref=ry_buf.at[i],
                send_sem=sy.at[i],
                recv_sem=ry_sem.at[i],
                device_id=(mx, py),
                device_id_type=pl.DeviceIdType.MESH,
            )

        def src_row(i):
            if i < N_F:
                return my * F_ROWS + i * R
            return D_BASE + (i - N_F) * R

        def rx_out_row(i):
            if i < N_F:
                return px * m_per + my * F_ROWS + i * R
            return px * m_per + D_BASE + (i - N_F) * R

        for i in range(N_X):
            send_buf[i, :, :] = x_ref[
                pl.ds(src_row(i), R), pl.ds(px * n_per, n_per)
            ].astype(jnp.bfloat16)
            x_rdma(i).start()

        out_ref[pl.ds(mx * m_per, m_per), :] = x_ref[:, pl.ds(mx * n_per, n_per)]

        Y_LAG = 6
        y_done = 0

        def drain_y(upto):
            nonlocal y_done
            while y_done < min(upto, N_F):
                i = y_done
                y_rdma(i).wait_recv()
                out_ref[pl.ds(px * m_per + py * F_ROWS + i * R, R), :] = ry_buf[
                    i
                ].astype(jnp.float32)
                y_done += 1

        for i in range(N_F):
            x_rdma(i).wait_recv()
            y_rdma(i).start()
            out_ref[pl.ds(rx_out_row(i), R), :] = rx_buf[i].astype(jnp.float32)
            drain_y(i - Y_LAG + 1)

        for j in range(N_D):
            i = N_F + j
            x_rdma(i).wait_recv()
            out_ref[pl.ds(rx_out_row(i), R), :] = rx_buf[i].astype(jnp.float32)
            drain_y(y_done + 1)
        drain_y(N_F)

        for i in range(N_X):
            x_rdma(i).wait_send()
        for i in range(N_F):
            y_rdma(i).wait_send()

    return pl.pallas_call(
        body,
        out_shape=jax.ShapeDtypeStruct((m_glob, n_per), x.dtype),
        in_specs=[pl.BlockSpec(memory_space=pltpu.VMEM)],
        out_specs=pl.BlockSpec(memory_space=pltpu.VMEM),
        scratch_shapes=[
            pltpu.VMEM((N_X, R, n_per), jnp.bfloat16),
            pltpu.VMEM((N_X, R, n_per), jnp.bfloat16),
            pltpu.VMEM((N_F, R, n_per), jnp.bfloat16),
            pltpu.SemaphoreType.DMA((N_X,)),
            pltpu.SemaphoreType.DMA((N_X,)),
            pltpu.SemaphoreType.DMA((N_F,)),
            pltpu.SemaphoreType.DMA((N_F,)),
        ],
        compiler_params=pltpu.CompilerParams(collective_id=0),
    )(x)
